# Initial kernel scaffold; baseline (speedup 1.0000x reference)
import jax
import jax.numpy as jnp
from jax.experimental import pallas as pl


def kernel(x, y):
    raise NotImplementedError("write your pallas kernel here")


# speedup vs baseline: 2.0108x; 2.0108x over previous
import functools

import jax
import jax.numpy as jnp
from jax.experimental import pallas as pl
from jax.experimental.pallas import tpu as pltpu

_SMOOTH = 0.1
_PAD = 0
_CONF = 1.0 - _SMOOTH


def _rowstats_kernel(x_ref, y_ref, out_ref, m_sc, s_sc, t_sc, g_sc, x0_sc,
                     *, C, Cb, n_cb):
    j = pl.program_id(1)
    xb = x_ref[...]
    rb = xb.shape[0]
    yb = y_ref[...]

    cols = jax.lax.broadcasted_iota(jnp.int32, xb.shape, 1) + j * Cb
    valid = cols < C
    xv = jnp.where(valid, xb, -jnp.inf)

    bmax = jnp.max(xv, axis=1, keepdims=True)
    bsum = jnp.sum(jnp.where(valid, xb, 0.0), axis=1, keepdims=True)
    bg = jnp.sum(jnp.where(cols == yb, xb, 0.0), axis=1, keepdims=True)

    @pl.when(j == 0)
    def _init():
        m_sc[...] = bmax
        s_sc[...] = jnp.sum(jnp.exp(xv - bmax), axis=1, keepdims=True)
        t_sc[...] = bsum
        g_sc[...] = bg
        x0_sc[...] = xb[:, 0:1]

    @pl.when(j > 0)
    def _acc():
        m_old = m_sc[...]
        m_new = jnp.maximum(m_old, bmax)
        s_sc[...] = (s_sc[...] * jnp.exp(m_old - m_new)
                     + jnp.sum(jnp.exp(xv - m_new), axis=1, keepdims=True))
        m_sc[...] = m_new
        t_sc[...] = t_sc[...] + bsum
        g_sc[...] = g_sc[...] + bg

    @pl.when(j == n_cb - 1)
    def _emit():
        eps = _SMOOTH / (C - 2)
        K = _SMOOTH * jnp.log(eps) + _CONF * jnp.log(_CONF)
        lse = m_sc[...] + jnp.log(s_sc[...])
        ssum = t_sc[...] - C * lse
        logp0 = x0_sc[...] - lse
        logpy = g_sc[...] - lse
        row = K - eps * (ssum - logp0 - logpy) - _CONF * logpy
        out_ref[...] = jnp.where(yb != _PAD, row, 0.0)


@jax.jit
def kernel(x, y):
    B, C = x.shape
    Rb, Cb = 512, 2048
    n_rb = B // Rb
    n_cb = pl.cdiv(C, Cb)
    y2 = y.astype(jnp.int32).reshape(B, 1)

    rows = pl.pallas_call(
        functools.partial(_rowstats_kernel, C=C, Cb=Cb, n_cb=n_cb),
        grid=(n_rb, n_cb),
        in_specs=[
            pl.BlockSpec((Rb, Cb), lambda i, j: (i, j)),
            pl.BlockSpec((Rb, 1), lambda i, j: (i, 0)),
        ],
        out_specs=pl.BlockSpec((Rb, 1), lambda i, j: (i, 0)),
        out_shape=jax.ShapeDtypeStruct((B, 1), x.dtype),
        scratch_shapes=[pltpu.VMEM((Rb, 1), jnp.float32)] * 5,
        compiler_params=pltpu.CompilerParams(
            dimension_semantics=("parallel", "arbitrary"),
        ),
    )(x, y2)
    return jnp.sum(rows)
